# baseline (speedup 1.0000x reference)
import functools

import jax
import jax.numpy as jnp
from jax import lax
from jax.experimental import pallas as pl
from jax.experimental.pallas import tpu as pltpu
from jax.experimental.pallas import tpu_sc as plsc

_R = 32
_L = 16


def _tokens_body(x_ref, tbits_ref):
    tokens = jnp.mean(jnp.abs(x_ref[...]), axis=1)
    tbits_ref[...] = lax.bitcast_convert_type(tokens, jnp.int32)


def _mul_body(x_ref, m_ref, pruned_ref):
    keepf = (m_ref[...] != 0).astype(jnp.float32)
    pruned_ref[...] = x_ref[...] * keepf[:, None, :]


def _make_sc_mask(n_rows, hw, keep_k):
    info = plsc.get_sparse_core_info()
    nc, ns = info.num_cores, info.num_subcores
    nw = nc * ns
    rows_pw = n_rows // nw
    n_sl = hw // _L
    mesh = plsc.VectorSubcoreMesh(core_axis_name="c", subcore_axis_name="s")

    @functools.partial(
        pl.kernel,
        mesh=mesh,
        out_type=jax.ShapeDtypeStruct((n_rows, hw), jnp.int32),
        scratch_types=[
            pltpu.VMEM((rows_pw, hw), jnp.int32),
            pltpu.VMEM((rows_pw, hw), jnp.int32),
        ],
        compiler_params=pltpu.CompilerParams(needs_layout_passes=False),
    )
    def sc_mask(tb_hbm, mask_hbm, tb_v, mask_v):
        wid = lax.axis_index("s") * nc + lax.axis_index("c")
        base = wid * rows_pw
        pltpu.sync_copy(tb_hbm.at[pl.ds(base, rows_pw)], tb_v)

        ones = jnp.ones((_L,), jnp.int32)

        for rr in range(rows_pw):
            def count_ge(mid):
                def sl_step(j, acc):
                    v = tb_v[rr, pl.ds(j * _L, _L)]
                    return acc + plsc.all_reduce_population_count(v >= mid)
                return lax.fori_loop(0, n_sl, sl_step,
                                     jnp.zeros((_L,), jnp.int32), unroll=16)

            def bs_step(_, carry):
                lo, hi = carry
                mid = lo + ((hi - lo + 1) >> 1)
                ge = count_ge(mid) >= keep_k
                return (jnp.where(ge, mid, lo),
                        jnp.where(ge, hi, mid - 1))

            thr, _ = lax.fori_loop(
                0, 31, bs_step,
                (jnp.zeros((_L,), jnp.int32),
                 jnp.full((_L,), 0x7F800000, jnp.int32)))

            c_gt = count_ge(thr + 1)
            need = keep_k - c_gt

            def write_step(j, tie_seen):
                v = tb_v[rr, pl.ds(j * _L, _L)]
                eq = v == thr
                csum = plsc.cumsum(jnp.where(eq, ones, 0))
                keep_tie = eq & ((tie_seen + csum) <= need)
                m = jnp.where((v > thr) | keep_tie, 1, 0)
                mask_v[rr, pl.ds(j * _L, _L)] = m
                return tie_seen + plsc.all_reduce_population_count(eq)

            lax.fori_loop(0, n_sl, write_step, jnp.zeros((_L,), jnp.int32),
                          unroll=8)

        pltpu.sync_copy(mask_v, mask_hbm.at[pl.ds(base, rows_pw)])

    return sc_mask


def kernel(x):
    B, T, C, H, W = x.shape
    BT, HW = B * T, H * W
    keep_k = max(1, int(HW * 0.5))
    x3 = x.reshape(BT, C, HW)

    tbits = pl.pallas_call(
        _tokens_body,
        grid=(BT // _R,),
        in_specs=[pl.BlockSpec((_R, C, HW), lambda i: (i, 0, 0))],
        out_specs=pl.BlockSpec((_R, HW), lambda i: (i, 0)),
        out_shape=jax.ShapeDtypeStruct((BT, HW), jnp.int32),
    )(x3)

    mask_i = _make_sc_mask(BT, HW, keep_k)(tbits)

    pruned3 = pl.pallas_call(
        _mul_body,
        grid=(BT // _R,),
        in_specs=[
            pl.BlockSpec((_R, C, HW), lambda i: (i, 0, 0)),
            pl.BlockSpec((_R, HW), lambda i: (i, 0)),
        ],
        out_specs=pl.BlockSpec((_R, C, HW), lambda i: (i, 0, 0)),
        out_shape=jax.ShapeDtypeStruct((BT, C, HW), x.dtype),
    )(x3, mask_i)

    pruned = pruned3.reshape(B, T, C, H, W)
    mask_2d = mask_i.astype(bool).reshape(B, T, H, W)
    return (pruned, mask_2d, mask_2d)


# speedup vs baseline: 1.0041x; 1.0041x over previous
import functools

import jax
import jax.numpy as jnp
from jax import lax
from jax.experimental import pallas as pl
from jax.experimental.pallas import tpu as pltpu
from jax.experimental.pallas import tpu_sc as plsc

_R = 32
_L = 16


def _tokens_body(x_ref, tbits_ref):
    tokens = jnp.mean(jnp.abs(x_ref[...]), axis=1)
    tbits_ref[...] = lax.bitcast_convert_type(tokens, jnp.int32)


def _mul_body(x_ref, m_ref, pruned_ref):
    keepf = (m_ref[...] != 0).astype(jnp.float32)
    pruned_ref[...] = x_ref[...] * keepf[:, None, :]


def _make_sc_mask(n_rows, hw, keep_k):
    info = plsc.get_sparse_core_info()
    nc, ns = info.num_cores, info.num_subcores
    nw = nc * ns
    rows_pw = n_rows // nw
    n_sl = hw // _L
    mesh = plsc.VectorSubcoreMesh(core_axis_name="c", subcore_axis_name="s")

    @functools.partial(
        pl.kernel,
        mesh=mesh,
        out_type=jax.ShapeDtypeStruct((n_rows, hw), jnp.int32),
        scratch_types=[
            pltpu.VMEM((rows_pw, hw), jnp.int32),
            pltpu.VMEM((rows_pw, hw), jnp.int32),
        ],
        compiler_params=pltpu.CompilerParams(needs_layout_passes=False),
    )
    def sc_mask(tb_hbm, mask_hbm, tb_v, mask_v):
        wid = lax.axis_index("s") * nc + lax.axis_index("c")
        base = wid * rows_pw
        pltpu.sync_copy(tb_hbm.at[pl.ds(base, rows_pw)], tb_v)

        ones = jnp.ones((_L,), jnp.int32)

        for rr in range(rows_pw):
            def count_ge(mid):
                def sl_step(j, acc):
                    v = tb_v[rr, pl.ds(j * _L, _L)]
                    return acc + plsc.all_reduce_population_count(v >= mid)
                return lax.fori_loop(0, n_sl, sl_step,
                                     jnp.zeros((_L,), jnp.int32), unroll=8)

            def bs_step(_, carry):
                lo, hi = carry
                mid = lo + ((hi - lo + 1) >> 1)
                ge = count_ge(mid) >= keep_k
                return (jnp.where(ge, mid, lo),
                        jnp.where(ge, hi, mid - 1))

            thr, _ = lax.fori_loop(
                0, 31, bs_step,
                (jnp.zeros((_L,), jnp.int32),
                 jnp.full((_L,), 0x7F800000, jnp.int32)))

            c_gt = count_ge(thr + 1)
            need = keep_k - c_gt

            def write_step(j, tie_seen):
                v = tb_v[rr, pl.ds(j * _L, _L)]
                eq = v == thr
                csum = plsc.cumsum(jnp.where(eq, ones, 0))
                keep_tie = eq & ((tie_seen + csum) <= need)
                m = jnp.where((v > thr) | keep_tie, 1, 0)
                mask_v[rr, pl.ds(j * _L, _L)] = m
                return tie_seen + plsc.all_reduce_population_count(eq)

            lax.fori_loop(0, n_sl, write_step, jnp.zeros((_L,), jnp.int32),
                          unroll=4)

        pltpu.sync_copy(mask_v, mask_hbm.at[pl.ds(base, rows_pw)])

    return sc_mask


def kernel(x):
    B, T, C, H, W = x.shape
    BT, HW = B * T, H * W
    keep_k = max(1, int(HW * 0.5))
    x3 = x.reshape(BT, C, HW)

    RT = 64
    tbits = pl.pallas_call(
        _tokens_body,
        grid=(BT // RT,),
        in_specs=[pl.BlockSpec((RT, C, HW), lambda i: (i, 0, 0))],
        out_specs=pl.BlockSpec((RT, HW), lambda i: (i, 0)),
        out_shape=jax.ShapeDtypeStruct((BT, HW), jnp.int32),
    )(x3)

    mask_i = _make_sc_mask(BT, HW, keep_k)(tbits)

    pruned3 = pl.pallas_call(
        _mul_body,
        grid=(BT // _R,),
        in_specs=[
            pl.BlockSpec((_R, C, HW), lambda i: (i, 0, 0)),
            pl.BlockSpec((_R, HW), lambda i: (i, 0)),
        ],
        out_specs=pl.BlockSpec((_R, C, HW), lambda i: (i, 0, 0)),
        out_shape=jax.ShapeDtypeStruct((BT, C, HW), x.dtype),
    )(x3, mask_i)

    pruned = pruned3.reshape(B, T, C, H, W)
    mask_2d = mask_i.astype(bool).reshape(B, T, H, W)
    return (pruned, mask_2d, mask_2d)


# speedup vs baseline: 1.0070x; 1.0030x over previous
import functools

import jax
import jax.numpy as jnp
from jax import lax
from jax.experimental import pallas as pl
from jax.experimental.pallas import tpu as pltpu
from jax.experimental.pallas import tpu_sc as plsc

_R = 32
_L = 16


def _tokens_body(x_ref, tbits_ref):
    tokens = jnp.mean(jnp.abs(x_ref[...]), axis=1)
    tbits_ref[...] = lax.bitcast_convert_type(tokens, jnp.int32)


def _mul_body(x_ref, m_ref, pruned_ref):
    keepf = (m_ref[...] != 0).astype(jnp.float32)
    pruned_ref[...] = x_ref[...] * keepf[:, None, :]


def _make_sc_mask(n_rows, hw, keep_k):
    info = plsc.get_sparse_core_info()
    nc, ns = info.num_cores, info.num_subcores
    nw = nc * ns
    rows_pw = n_rows // nw
    n_sl = hw // _L
    mesh = plsc.VectorSubcoreMesh(core_axis_name="c", subcore_axis_name="s")

    @functools.partial(
        pl.kernel,
        mesh=mesh,
        out_type=jax.ShapeDtypeStruct((n_rows, hw), jnp.int32),
        scratch_types=[
            pltpu.VMEM((rows_pw, hw), jnp.int32),
            pltpu.VMEM((rows_pw, hw), jnp.int32),
        ],
        compiler_params=pltpu.CompilerParams(needs_layout_passes=False),
    )
    def sc_mask(tb_hbm, mask_hbm, tb_v, mask_v):
        wid = lax.axis_index("s") * nc + lax.axis_index("c")
        base = wid * rows_pw
        pltpu.sync_copy(tb_hbm.at[pl.ds(base, rows_pw)], tb_v)

        ones = jnp.ones((_L,), jnp.int32)

        for rr in range(rows_pw):
            def count_ge(mid):
                def sl_step(j, acc):
                    v = tb_v[rr, pl.ds(j * _L, _L)]
                    return acc + plsc.all_reduce_population_count(v >= mid)
                return lax.fori_loop(0, n_sl, sl_step,
                                     jnp.zeros((_L,), jnp.int32), unroll=8)

            def bs_step(_, carry):
                lo, hi = carry
                mid = lo + ((hi - lo + 1) >> 1)
                ge = count_ge(mid) >= keep_k
                return (jnp.where(ge, mid, lo),
                        jnp.where(ge, hi, mid - 1))

            thr, _ = lax.fori_loop(
                0, 31, bs_step,
                (jnp.zeros((_L,), jnp.int32),
                 jnp.full((_L,), 0x7F800000, jnp.int32)))

            c_gt = count_ge(thr + 1)
            need = keep_k - c_gt

            def write_step(j, tie_seen):
                v = tb_v[rr, pl.ds(j * _L, _L)]
                eq = v == thr
                csum = plsc.cumsum(jnp.where(eq, ones, 0))
                keep_tie = eq & ((tie_seen + csum) <= need)
                m = jnp.where((v > thr) | keep_tie, 1, 0)
                mask_v[rr, pl.ds(j * _L, _L)] = m
                return tie_seen + plsc.all_reduce_population_count(eq)

            lax.fori_loop(0, n_sl, write_step, jnp.zeros((_L,), jnp.int32),
                          unroll=4)

        pltpu.sync_copy(mask_v, mask_hbm.at[pl.ds(base, rows_pw)])

    return sc_mask


def kernel(x):
    B, T, C, H, W = x.shape
    BT, HW = B * T, H * W
    keep_k = max(1, int(HW * 0.5))
    x3 = x.reshape(BT, C, HW)

    half = BT // 2
    sc_mask = _make_sc_mask(half, HW, keep_k)

    def tok_chunk(off):
        return pl.pallas_call(
            _tokens_body,
            grid=(half // _R,),
            in_specs=[pl.BlockSpec((_R, C, HW), lambda i, o=off: (i + o, 0, 0))],
            out_specs=pl.BlockSpec((_R, HW), lambda i: (i, 0)),
            out_shape=jax.ShapeDtypeStruct((half, HW), jnp.int32),
        )(x3)

    mask_a = sc_mask(tok_chunk(0))
    mask_b = sc_mask(tok_chunk(half // _R))
    mask_i = jnp.concatenate([mask_a, mask_b], axis=0)

    pruned3 = pl.pallas_call(
        _mul_body,
        grid=(BT // _R,),
        in_specs=[
            pl.BlockSpec((_R, C, HW), lambda i: (i, 0, 0)),
            pl.BlockSpec((_R, HW), lambda i: (i, 0)),
        ],
        out_specs=pl.BlockSpec((_R, C, HW), lambda i: (i, 0, 0)),
        out_shape=jax.ShapeDtypeStruct((BT, C, HW), x.dtype),
    )(x3, mask_i)

    pruned = pruned3.reshape(B, T, C, H, W)
    mask_2d = mask_i.astype(bool).reshape(B, T, H, W)
    return (pruned, mask_2d, mask_2d)


# speedup vs baseline: 1.0085x; 1.0014x over previous
import functools

import jax
import jax.numpy as jnp
from jax import lax
from jax.experimental import pallas as pl
from jax.experimental.pallas import tpu as pltpu
from jax.experimental.pallas import tpu_sc as plsc

_R = 32
_L = 16


def _tokens_body(x_ref, tbits_ref):
    tokens = jnp.mean(jnp.abs(x_ref[...]), axis=1)
    tbits_ref[...] = lax.bitcast_convert_type(tokens, jnp.int32)


def _mul_body(x_ref, m_ref, pruned_ref):
    keepf = (m_ref[...] != 0).astype(jnp.float32)
    pruned_ref[...] = x_ref[...] * keepf[:, None, :]


def _make_sc_mask(n_rows, hw, keep_k):
    info = plsc.get_sparse_core_info()
    nc, ns = info.num_cores, info.num_subcores
    nw = nc * ns
    rows_pw = n_rows // nw
    n_sl = hw // _L
    mesh = plsc.VectorSubcoreMesh(core_axis_name="c", subcore_axis_name="s")

    @functools.partial(
        pl.kernel,
        mesh=mesh,
        out_type=jax.ShapeDtypeStruct((n_rows, hw), jnp.int32),
        scratch_types=[
            pltpu.VMEM((rows_pw, hw), jnp.int32),
            pltpu.VMEM((rows_pw, hw), jnp.int32),
        ],
        compiler_params=pltpu.CompilerParams(needs_layout_passes=False),
    )
    def sc_mask(tb_hbm, mask_hbm, tb_v, mask_v):
        wid = lax.axis_index("s") * nc + lax.axis_index("c")
        base = wid * rows_pw
        pltpu.sync_copy(tb_hbm.at[pl.ds(base, rows_pw)], tb_v)

        ones = jnp.ones((_L,), jnp.int32)

        for rr in range(rows_pw):
            def count_ge(mid):
                def sl_step(j, acc):
                    v = tb_v[rr, pl.ds(j * _L, _L)]
                    return acc + plsc.all_reduce_population_count(v >= mid)
                return lax.fori_loop(0, n_sl, sl_step,
                                     jnp.zeros((_L,), jnp.int32), unroll=8)

            def bs_step(_, carry):
                lo, hi = carry
                mid = lo + ((hi - lo + 1) >> 1)
                ge = count_ge(mid) >= keep_k
                return (jnp.where(ge, mid, lo),
                        jnp.where(ge, hi, mid - 1))

            thr, _ = lax.fori_loop(
                0, 31, bs_step,
                (jnp.zeros((_L,), jnp.int32),
                 jnp.full((_L,), 0x7F800000, jnp.int32)))

            c_gt = count_ge(thr + 1)
            need = keep_k - c_gt

            def write_step(j, tie_seen):
                v = tb_v[rr, pl.ds(j * _L, _L)]
                eq = v == thr
                csum = plsc.cumsum(jnp.where(eq, ones, 0))
                keep_tie = eq & ((tie_seen + csum) <= need)
                m = jnp.where((v > thr) | keep_tie, 1, 0)
                mask_v[rr, pl.ds(j * _L, _L)] = m
                return tie_seen + plsc.all_reduce_population_count(eq)

            lax.fori_loop(0, n_sl, write_step, jnp.zeros((_L,), jnp.int32),
                          unroll=4)

        pltpu.sync_copy(mask_v, mask_hbm.at[pl.ds(base, rows_pw)])

    return sc_mask


def kernel(x):
    B, T, C, H, W = x.shape
    BT, HW = B * T, H * W
    keep_k = max(1, int(HW * 0.5))
    x3 = x.reshape(BT, C, HW)

    tbits = pl.pallas_call(
        _tokens_body,
        grid=(BT // _R,),
        in_specs=[pl.BlockSpec((_R, C, HW), lambda i: (i, 0, 0))],
        out_specs=pl.BlockSpec((_R, HW), lambda i: (i, 0)),
        out_shape=jax.ShapeDtypeStruct((BT, HW), jnp.int32),
    )(x3)

    mask_i = _make_sc_mask(BT, HW, keep_k)(tbits)

    pruned3 = pl.pallas_call(
        _mul_body,
        grid=(BT // _R,),
        in_specs=[
            pl.BlockSpec((_R, C, HW), lambda i: (i, 0, 0)),
            pl.BlockSpec((_R, HW), lambda i: (i, 0)),
        ],
        out_specs=pl.BlockSpec((_R, C, HW), lambda i: (i, 0, 0)),
        out_shape=jax.ShapeDtypeStruct((BT, C, HW), x.dtype),
    )(x3, mask_i)

    pruned = pruned3.reshape(B, T, C, H, W)
    mask_2d = mask_i.astype(bool).reshape(B, T, H, W)
    return (pruned, mask_2d, mask_2d)
